# baseline (speedup 1.0000x reference)
import functools

import jax
import jax.numpy as jnp
from jax import lax
from jax.experimental import pallas as pl
from jax.experimental.pallas import tpu as pltpu
from jax.experimental.pallas import tpu_sc as plsc

N_NODES = 10000
N_EDGES = 320000
D = 128
DEGW = 128

NC = 2
NS = 16
NW = NC * NS
E_PER_W = N_EDGES // NW
K = 25
G = 8
NG = E_PER_W // (G * K)
TOT = NG * G
N_PAD = 10240
ROWS_PER_TILE = N_PAD // NS


def _sc_agg_body(d, nbuf, *refs):
    pf = 2
    dd = nbuf - pf
    x_hbm, srcG_hbm, dstG_hbm, z_hbm, acc_out = refs[:5]
    srcT, dstT, acc_sh = refs[5:8]
    rows = refs[8:8 + nbuf]
    gsem = refs[8 + nbuf:8 + 2 * nbuf]
    ssem = refs[8 + 2 * nbuf:8 + 3 * nbuf]
    tsem = refs[8 + 3 * nbuf]

    cid = lax.axis_index("c")
    sid = lax.axis_index("s")
    wid = sid * NC + cid

    r0 = sid * ROWS_PER_TILE
    pltpu.sync_copy(z_hbm, acc_sh.at[pl.ds(r0, ROWS_PER_TILE)])

    plsc.subcore_barrier()

    pltpu.sync_copy(srcG_hbm.at[wid, 0], srcT.at[0])
    pltpu.sync_copy(dstG_hbm.at[wid, 0], dstT.at[0])
    pltpu.async_copy(srcG_hbm.at[wid, 1], srcT.at[1], tsem)
    pltpu.async_copy(dstG_hbm.at[wid, 1], dstT.at[1], tsem)
    for j in range(pf):
        pltpu.async_copy(x_hbm.at[srcT.at[0, j]], rows[j], gsem[j])

    def outer(grp, _):
        h = lax.rem(grp, 3)
        h1 = lax.rem(grp + 1, 3)
        h2 = lax.rem(grp + 2, 3)
        for j in range(G):
            b = j % nbuf
            c = grp * G + j
            pltpu.make_async_copy(
                x_hbm.at[srcT.at[h, j]], rows[b], gsem[b]).wait()
            pltpu.async_copy(rows[b], acc_sh.at[dstT.at[h, j]], ssem[b],
                             add=True)
            if j == G - pf - 1:
                @pl.when(grp + 1 < NG)
                def _():
                    pltpu.make_async_copy(
                        srcG_hbm.at[wid, 0], srcT.at[0], tsem).wait()
                    pltpu.make_async_copy(
                        dstG_hbm.at[wid, 0], dstT.at[0], tsem).wait()
            if j == G - pf:
                @pl.when(grp + 2 < NG)
                def _():
                    pltpu.async_copy(srcG_hbm.at[wid, grp + 2],
                                     srcT.at[h2], tsem)
                    pltpu.async_copy(dstG_hbm.at[wid, grp + 2],
                                     dstT.at[h2], tsem)
            b2 = (j + pf) % nbuf

            @pl.when(c >= dd)
            def _():
                pltpu.make_async_copy(
                    rows[b2], acc_sh.at[dstT.at[h, j]], ssem[b2]).wait()

            @pl.when(c + pf < TOT)
            def _():
                if j + pf < G:
                    pltpu.async_copy(x_hbm.at[srcT.at[h, j + pf]], rows[b2],
                                     gsem[b2])
                else:
                    pltpu.async_copy(x_hbm.at[srcT.at[h1, j + pf - G]],
                                     rows[b2], gsem[b2])
        return 0
    lax.fori_loop(0, NG, outer, 0)

    for c in range(TOT - dd, TOT):
        b = c % nbuf
        pltpu.make_async_copy(rows[b], acc_sh.at[dstT.at[0, 0]],
                              ssem[b]).wait()

    plsc.subcore_barrier()

    pltpu.sync_copy(acc_sh.at[pl.ds(r0, ROWS_PER_TILE)],
                    acc_out.at[cid, pl.ds(r0, ROWS_PER_TILE)])


def _make_sc_agg(d, nbuf):
    mesh = plsc.VectorSubcoreMesh(core_axis_name="c", subcore_axis_name="s",
                                  num_cores=NC, num_subcores=NS)
    return pl.kernel(
        functools.partial(_sc_agg_body, d, nbuf),
        out_type=[jax.ShapeDtypeStruct((NC, N_PAD, d), jnp.float32)],
        mesh=mesh,
        scratch_types=(
            [
                pltpu.VMEM((3, G, K), jnp.int32),
                pltpu.VMEM((3, G, K), jnp.int32),
                pltpu.VMEM_SHARED((N_PAD, d), jnp.float32),
            ]
            + [pltpu.VMEM((K, d), jnp.float32) for _ in range(nbuf)]
            + [pltpu.SemaphoreType.DMA for _ in range(2 * nbuf + 1)]
        ),
    )


_sc_agg = _make_sc_agg(D, 8)


def _sc_deg_body(dst_hbm, deg_out, dstL, hist):
    cid = lax.axis_index("c")
    sid = lax.axis_index("s")
    wid = sid * NC + cid

    zero16 = jnp.zeros((16,), jnp.float32)

    def zrow(i, _):
        hist[pl.ds(i * 16, 16)] = zero16
        return 0
    lax.fori_loop(0, N_PAD // 16, zrow, 0)

    pltpu.sync_copy(dst_hbm.at[pl.ds(wid * E_PER_W, E_PER_W)], dstL)

    one16 = jnp.ones((16,), jnp.float32)

    def step(i, _):
        iv = dstL[pl.ds(i * 16, 16)]
        plsc.addupdate_scatter(hist, [iv], one16)
        return 0
    lax.fori_loop(0, E_PER_W // 16, step, 0)

    pltpu.sync_copy(hist, deg_out.at[cid, sid])


def _make_sc_deg():
    mesh = plsc.VectorSubcoreMesh(core_axis_name="c", subcore_axis_name="s",
                                  num_cores=NC, num_subcores=NS)
    return pl.kernel(
        _sc_deg_body,
        out_type=[jax.ShapeDtypeStruct((NC, NS, N_PAD), jnp.float32)],
        mesh=mesh,
        compiler_params=pltpu.CompilerParams(needs_layout_passes=False),
        scratch_types=[
            pltpu.VMEM((E_PER_W,), jnp.int32),
            pltpu.VMEM((N_PAD,), jnp.float32),
        ],
    )


_sc_deg = _make_sc_deg()


R_BLK = 2048


def _tc_dense_body(dacc, prelu, final, *refs):
    if final:
        (acc_ref, deg_ref, h_ref, wl_ref, bl_ref, wr_ref, a_ref,
         wp_ref, bp_ref, out_ref) = refs
    else:
        (acc_ref, deg_ref, h_ref, wl_ref, bl_ref, wr_ref, a_ref,
         out_ref) = refs
    acc = acc_ref[0] + acc_ref[1]
    deg = jnp.sum(deg_ref[...], axis=0)
    mean = acc * (1.0 / jnp.clip(deg, 1.0, None))[:, None]
    h = h_ref[...]
    out = (jnp.dot(mean, wl_ref[...], preferred_element_type=jnp.float32)
           + bl_ref[...][None, :]
           + jnp.dot(h, wr_ref[...], preferred_element_type=jnp.float32))
    if prelu:
        a = a_ref[0, 0]
        out = jnp.where(out >= 0, out, a * out)
    if final:
        lvl = jnp.dot(out, wp_ref[...], preferred_element_type=jnp.float32)
        out_ref[...] = lvl + bp_ref[...][None, :]
    else:
        out_ref[...] = out


def _make_tc_dense(dacc, prelu, final):
    n_blk = N_PAD // R_BLK
    full = lambda i: (0, 0)
    in_specs = [
        pl.BlockSpec((NC, R_BLK, dacc), lambda i: (0, i, 0)),
        pl.BlockSpec((NW, R_BLK), lambda i: (0, i)),
        pl.BlockSpec((R_BLK, D), lambda i: (i, 0)),
        pl.BlockSpec((D, D), full),
        pl.BlockSpec((D,), lambda i: (0,)),
        pl.BlockSpec((D, D), full),
        pl.BlockSpec((1, 1), full),
    ]
    if final:
        in_specs += [
            pl.BlockSpec((D, 1), full),
            pl.BlockSpec((1,), lambda i: (0,)),
        ]
        out_spec = pl.BlockSpec((R_BLK, 1), lambda i: (i, 0))
        out_shape = jax.ShapeDtypeStruct((N_PAD, 1), jnp.float32)
    else:
        out_spec = pl.BlockSpec((R_BLK, D), lambda i: (i, 0))
        out_shape = jax.ShapeDtypeStruct((N_PAD, D), jnp.float32)
    return pl.pallas_call(
        functools.partial(_tc_dense_body, dacc, prelu, final),
        grid=(n_blk,),
        in_specs=in_specs,
        out_specs=out_spec,
        out_shape=out_shape,
    )


_tc_mid = _make_tc_dense(D, True, False)
_tc_last = _make_tc_dense(D, False, True)


def kernel(x, edge_index, Wl1, bl1, Wr1, Wl2, bl2, Wr2, Wl3, bl3, Wr3,
           a, Wp, bp):
    srcG = edge_index[0].astype(jnp.int32).reshape(NW, NG, G, K)
    dstG = edge_index[1].astype(jnp.int32).reshape(NW, NG, G, K)
    a2 = jnp.asarray(a, jnp.float32).reshape(1, 1)
    z = jnp.zeros((ROWS_PER_TILE, D), jnp.float32)
    xp = jnp.pad(x, ((0, N_PAD - N_NODES), (0, 0)))

    dst_flat = edge_index[1].astype(jnp.int32)
    degp, = _sc_deg(dst_flat)
    degp = degp.reshape(NW, N_PAD)
    acc1, = _sc_agg(xp, srcG, dstG, z)
    h1 = _tc_mid(acc1, degp, xp, Wl1, bl1, Wr1, a2)
    acc2, = _sc_agg(h1, srcG, dstG, z)
    h2 = _tc_mid(acc2, degp, h1, Wl2, bl2, Wr2, a2)
    acc3, = _sc_agg(h2, srcG, dstG, z)
    out = _tc_last(acc3, degp, h2, Wl3, bl3, Wr3, a2, Wp, bp)
    return out[:N_NODES, 0]


# speedup vs baseline: 1.3539x; 1.3539x over previous
import functools

import jax
import jax.numpy as jnp
from jax import lax
from jax.experimental import pallas as pl
from jax.experimental.pallas import tpu as pltpu
from jax.experimental.pallas import tpu_sc as plsc

N_NODES = 10000
N_EDGES = 320000
D = 128
DEGW = 128

NC = 2
NS = 16
NW = NC * NS
E_PER_W = N_EDGES // NW
K = 25
G = 8
NG = E_PER_W // (G * K)
TOT = NG * G
N_PAD = 10240
ROWS_PER_TILE = N_PAD // NS


def _sc_agg_body(d, nbuf, *refs):
    pf = nbuf // 2
    x_hbm, srcG_hbm, dstG_hbm, z_hbm, acc_out = refs[:5]
    srcT, dstT, acc_sh = refs[5:8]
    rows = refs[8:8 + nbuf]
    gsem = refs[8 + nbuf:8 + 2 * nbuf]
    ssem = refs[8 + 2 * nbuf:8 + 3 * nbuf]
    tsem = refs[8 + 3 * nbuf]

    cid = lax.axis_index("c")
    sid = lax.axis_index("s")
    wid = sid * NC + cid

    r0 = sid * ROWS_PER_TILE
    pltpu.sync_copy(z_hbm, acc_sh.at[pl.ds(r0, ROWS_PER_TILE)])

    plsc.subcore_barrier()

    pltpu.sync_copy(srcG_hbm.at[wid, 0], srcT.at[0])
    pltpu.sync_copy(dstG_hbm.at[wid, 0], dstT.at[0])
    pltpu.async_copy(srcG_hbm.at[wid, 1], srcT.at[1], tsem)
    pltpu.async_copy(dstG_hbm.at[wid, 1], dstT.at[1], tsem)
    for j in range(pf):
        pltpu.async_copy(x_hbm.at[srcT.at[0, j]], rows[j], gsem[j])

    def outer(grp, _):
        h = lax.rem(grp, 3)
        h1 = lax.rem(grp + 1, 3)
        h2 = lax.rem(grp + 2, 3)
        for j in range(G):
            b = j % nbuf
            c = grp * G + j
            pltpu.make_async_copy(
                x_hbm.at[srcT.at[h, j]], rows[b], gsem[b]).wait()
            pltpu.async_copy(rows[b], acc_sh.at[dstT.at[h, j]], ssem[b],
                             add=True)
            if j == 3:
                @pl.when(grp + 1 < NG)
                def _():
                    pltpu.make_async_copy(
                        srcG_hbm.at[wid, 0], srcT.at[0], tsem).wait()
                    pltpu.make_async_copy(
                        dstG_hbm.at[wid, 0], dstT.at[0], tsem).wait()
            if j == 4:
                @pl.when(grp + 2 < NG)
                def _():
                    pltpu.async_copy(srcG_hbm.at[wid, grp + 2],
                                     srcT.at[h2], tsem)
                    pltpu.async_copy(dstG_hbm.at[wid, grp + 2],
                                     dstT.at[h2], tsem)
            b2 = (j + pf) % nbuf

            @pl.when(c >= pf)
            def _():
                pltpu.make_async_copy(
                    rows[b2], acc_sh.at[dstT.at[h, j]], ssem[b2]).wait()

            @pl.when(c + pf < TOT)
            def _():
                if j + pf < G:
                    pltpu.async_copy(x_hbm.at[srcT.at[h, j + pf]], rows[b2],
                                     gsem[b2])
                else:
                    pltpu.async_copy(x_hbm.at[srcT.at[h1, j + pf - G]],
                                     rows[b2], gsem[b2])
        return 0
    lax.fori_loop(0, NG, outer, 0)

    for c in range(TOT - pf, TOT):
        b = c % nbuf
        pltpu.make_async_copy(rows[b], acc_sh.at[dstT.at[0, 0]],
                              ssem[b]).wait()

    plsc.subcore_barrier()

    pltpu.sync_copy(acc_sh.at[pl.ds(r0, ROWS_PER_TILE)],
                    acc_out.at[cid, pl.ds(r0, ROWS_PER_TILE)])


def _make_sc_agg(d, nbuf):
    mesh = plsc.VectorSubcoreMesh(core_axis_name="c", subcore_axis_name="s",
                                  num_cores=NC, num_subcores=NS)
    return pl.kernel(
        functools.partial(_sc_agg_body, d, nbuf),
        out_type=[jax.ShapeDtypeStruct((NC, N_PAD, d), jnp.float32)],
        mesh=mesh,
        scratch_types=(
            [
                pltpu.VMEM((3, G, K), jnp.int32),
                pltpu.VMEM((3, G, K), jnp.int32),
                pltpu.VMEM_SHARED((N_PAD, d), jnp.float32),
            ]
            + [pltpu.VMEM((K, d), jnp.float32) for _ in range(nbuf)]
            + [pltpu.SemaphoreType.DMA for _ in range(2 * nbuf + 1)]
        ),
    )


_sc_agg = _make_sc_agg(D, 8)


def _sc_deg_body(dst_hbm, deg_out, dstL, hist):
    cid = lax.axis_index("c")
    sid = lax.axis_index("s")
    wid = sid * NC + cid

    zero16 = jnp.zeros((16,), jnp.float32)

    def zrow(i, _):
        hist[pl.ds(i * 16, 16)] = zero16
        return 0
    lax.fori_loop(0, N_PAD // 16, zrow, 0)

    pltpu.sync_copy(dst_hbm.at[pl.ds(wid * E_PER_W, E_PER_W)], dstL)

    one16 = jnp.ones((16,), jnp.float32)

    def step(i, _):
        iv = dstL[pl.ds(i * 16, 16)]
        plsc.addupdate_scatter(hist, [iv], one16)
        return 0
    lax.fori_loop(0, E_PER_W // 16, step, 0)

    pltpu.sync_copy(hist, deg_out.at[cid, sid])


def _make_sc_deg():
    mesh = plsc.VectorSubcoreMesh(core_axis_name="c", subcore_axis_name="s",
                                  num_cores=NC, num_subcores=NS)
    return pl.kernel(
        _sc_deg_body,
        out_type=[jax.ShapeDtypeStruct((NC, NS, N_PAD), jnp.float32)],
        mesh=mesh,
        compiler_params=pltpu.CompilerParams(needs_layout_passes=False),
        scratch_types=[
            pltpu.VMEM((E_PER_W,), jnp.int32),
            pltpu.VMEM((N_PAD,), jnp.float32),
        ],
    )


_sc_deg = _make_sc_deg()


R_BLK = 2048


def _tc_dense_body(dacc, prelu, final, *refs):
    if final:
        (acc_ref, deg_ref, h_ref, wl_ref, bl_ref, wr_ref, a_ref,
         wp_ref, bp_ref, out_ref) = refs
    else:
        (acc_ref, deg_ref, h_ref, wl_ref, bl_ref, wr_ref, a_ref,
         out_ref) = refs
    acc = acc_ref[0] + acc_ref[1]
    deg = jnp.sum(deg_ref[...], axis=0)
    mean = acc * (1.0 / jnp.clip(deg, 1.0, None))[:, None]
    h = h_ref[...]
    out = (jnp.dot(mean, wl_ref[...], preferred_element_type=jnp.float32)
           + bl_ref[...][None, :]
           + jnp.dot(h, wr_ref[...], preferred_element_type=jnp.float32))
    if prelu:
        a = a_ref[0, 0]
        out = jnp.where(out >= 0, out, a * out)
    if final:
        lvl = jnp.dot(out, wp_ref[...], preferred_element_type=jnp.float32)
        out_ref[...] = lvl + bp_ref[...][None, :]
    else:
        out_ref[...] = out


def _make_tc_dense(dacc, prelu, final):
    n_blk = N_PAD // R_BLK
    full = lambda i: (0, 0)
    in_specs = [
        pl.BlockSpec((NC, R_BLK, dacc), lambda i: (0, i, 0)),
        pl.BlockSpec((NW, R_BLK), lambda i: (0, i)),
        pl.BlockSpec((R_BLK, D), lambda i: (i, 0)),
        pl.BlockSpec((D, D), full),
        pl.BlockSpec((D,), lambda i: (0,)),
        pl.BlockSpec((D, D), full),
        pl.BlockSpec((1, 1), full),
    ]
    if final:
        in_specs += [
            pl.BlockSpec((D, 1), full),
            pl.BlockSpec((1,), lambda i: (0,)),
        ]
        out_spec = pl.BlockSpec((R_BLK, 1), lambda i: (i, 0))
        out_shape = jax.ShapeDtypeStruct((N_PAD, 1), jnp.float32)
    else:
        out_spec = pl.BlockSpec((R_BLK, D), lambda i: (i, 0))
        out_shape = jax.ShapeDtypeStruct((N_PAD, D), jnp.float32)
    return pl.pallas_call(
        functools.partial(_tc_dense_body, dacc, prelu, final),
        grid=(n_blk,),
        in_specs=in_specs,
        out_specs=out_spec,
        out_shape=out_shape,
    )


_tc_mid = _make_tc_dense(D, True, False)
_tc_last = _make_tc_dense(D, False, True)


def kernel(x, edge_index, Wl1, bl1, Wr1, Wl2, bl2, Wr2, Wl3, bl3, Wr3,
           a, Wp, bp):
    srcG = edge_index[0].astype(jnp.int32).reshape(NW, NG, G, K)
    dstG = edge_index[1].astype(jnp.int32).reshape(NW, NG, G, K)
    a2 = jnp.asarray(a, jnp.float32).reshape(1, 1)
    z = jnp.zeros((ROWS_PER_TILE, D), jnp.float32)
    xp = jnp.pad(x, ((0, N_PAD - N_NODES), (0, 0)))

    dst_flat = edge_index[1].astype(jnp.int32)
    degp, = _sc_deg(dst_flat)
    degp = degp.reshape(NW, N_PAD)
    acc1, = _sc_agg(xp, srcG, dstG, z)
    h1 = _tc_mid(acc1, degp, xp, Wl1, bl1, Wr1, a2)
    acc2, = _sc_agg(h1, srcG, dstG, z)
    h2 = _tc_mid(acc2, degp, h1, Wl2, bl2, Wr2, a2)
    acc3, = _sc_agg(h2, srcG, dstG, z)
    out = _tc_last(acc3, degp, h2, Wl3, bl3, Wr3, a2, Wp, bp)
    return out[:N_NODES, 0]
